# Initial kernel scaffold; baseline (speedup 1.0000x reference)
import jax
import jax.numpy as jnp
from jax.experimental import pallas as pl


def kernel(gates, inp, W, b):
    raise NotImplementedError("write your pallas kernel here")


# speedup vs baseline: 18.0068x; 18.0068x over previous
import functools

import jax
import jax.numpy as jnp
import numpy as np
from jax.experimental import pallas as pl

B = 32768
E = 8
D = 768
OUT = 64
TB = 512


def _moe_block_kernel(gates_ref, inp_ref, w2_ref, b2_ref, out_ref):
    x = inp_ref[...]
    g = gates_ref[...]
    y = jnp.dot(x, w2_ref[...], preferred_element_type=jnp.float32)
    y = y + b2_ref[...]
    z = jnp.exp(y)
    acc = jnp.zeros((x.shape[0], OUT), dtype=jnp.float32)
    for e in range(E):
        acc = acc + z[:, e * OUT:(e + 1) * OUT] * g[:, e][:, None]
    eps = jnp.float32(np.finfo(np.float64).eps)
    acc = jnp.where(acc == 0, eps, acc)
    out_ref[...] = jnp.log(acc)


@jax.jit
def kernel(gates, inp, W, b):
    w2 = W.transpose(1, 0, 2).reshape(D, E * OUT)
    b2 = b.reshape(1, E * OUT)
    grid = (B // TB,)
    return pl.pallas_call(
        _moe_block_kernel,
        grid=grid,
        in_specs=[
            pl.BlockSpec((TB, E), lambda i: (i, 0)),
            pl.BlockSpec((TB, D), lambda i: (i, 0)),
            pl.BlockSpec((D, E * OUT), lambda i: (0, 0)),
            pl.BlockSpec((1, E * OUT), lambda i: (0, 0)),
        ],
        out_specs=pl.BlockSpec((TB, OUT), lambda i: (i, 0)),
        out_shape=jax.ShapeDtypeStruct((B, OUT), jnp.float32),
    )(gates, inp, w2, b2)


# speedup vs baseline: 18.2112x; 1.0114x over previous
import functools

import jax
import jax.numpy as jnp
import numpy as np
from jax.experimental import pallas as pl

B = 32768
E = 8
D = 768
OUT = 64
TB = 512


def _moe_block_kernel(gates_ref, inp_ref, w2_ref, b2_ref, out_ref):
    x = inp_ref[...]
    g = gates_ref[...]
    y = jnp.dot(x.astype(jnp.bfloat16), w2_ref[...].astype(jnp.bfloat16),
                preferred_element_type=jnp.float32)
    y = y + b2_ref[...]
    z = jnp.exp(y)
    acc = jnp.zeros((x.shape[0], OUT), dtype=jnp.float32)
    for e in range(E):
        acc = acc + z[:, e * OUT:(e + 1) * OUT] * g[:, e][:, None]
    eps = jnp.float32(np.finfo(np.float64).eps)
    acc = jnp.where(acc == 0, eps, acc)
    out_ref[...] = jnp.log(acc)


@jax.jit
def kernel(gates, inp, W, b):
    w2 = W.transpose(1, 0, 2).reshape(D, E * OUT)
    b2 = b.reshape(1, E * OUT)
    grid = (B // TB,)
    return pl.pallas_call(
        _moe_block_kernel,
        grid=grid,
        in_specs=[
            pl.BlockSpec((TB, E), lambda i: (i, 0)),
            pl.BlockSpec((TB, D), lambda i: (i, 0)),
            pl.BlockSpec((D, E * OUT), lambda i: (0, 0)),
            pl.BlockSpec((1, E * OUT), lambda i: (0, 0)),
        ],
        out_specs=pl.BlockSpec((TB, OUT), lambda i: (i, 0)),
        out_shape=jax.ShapeDtypeStruct((B, OUT), jnp.float32),
    )(gates, inp, w2, b2)
